# baseline (speedup 1.0000x reference)
import functools

import jax
import jax.numpy as jnp
from jax import lax
from jax.experimental import pallas as pl
from jax.experimental.pallas import tpu as pltpu
from jax.experimental.pallas import tpu_sc as plsc

NN = 100000
EE = 3200000
NFD = 18
EFD = 3
NLAYER = 10

PADF = 32
MSGW = 8
G = 4
E4 = EE // G

NC, NS = 2, 16
NW = NC * NS
CH = 128
NCHUNK = EE // CH
RPT = NN // NS
ZR = 1250

NBLK = 1000
EB4 = 800

_mesh = plsc.VectorSubcoreMesh(core_axis_name="c", subcore_axis_name="s",
                               num_cores=NC, num_subcores=NS)
_sc_params = pltpu.CompilerParams(use_tc_tiling_on_sc=False)
_sc_params_nl = pltpu.CompilerParams(use_tc_tiling_on_sc=False,
                                     needs_layout_passes=False)

NB = 8


@functools.lru_cache(maxsize=None)
def _make_sc_gather(nchunk):
    ngrp = (nchunk + NW * NB - 1) // (NW * NB)

    @functools.partial(
        pl.kernel,
        out_type=(jax.ShapeDtypeStruct((nchunk * CH, PADF), jnp.float32),
                  jax.ShapeDtypeStruct((nchunk * CH, PADF), jnp.float32)),
        mesh=_mesh,
        scratch_types=[
            pltpu.VMEM((NB, CH), jnp.int32),
            pltpu.VMEM((NB, CH), jnp.int32),
            pltpu.VMEM((NB, CH, PADF), jnp.float32),
            pltpu.VMEM((NB, CH, PADF), jnp.float32),
            pltpu.SemaphoreType.DMA((NB,)),
            pltpu.SemaphoreType.DMA((NB,)),
            pltpu.SemaphoreType.DMA((NB,)),
        ],
        compiler_params=_sc_params,
    )
    def _sc_gather(node_hbm, dst2_hbm, src2_hbm, xi_hbm, xj_hbm,
                   di_v, si_v, ri_v, rj_v, sem_ix, sem_g, sem_wb):
        wid = lax.axis_index("s") * NC + lax.axis_index("c")

        def _wb_pair(b, c):
            return (pltpu.make_async_copy(ri_v.at[b],
                                          xi_hbm.at[pl.ds(c * CH, CH)],
                                          sem_wb.at[b]),
                    pltpu.make_async_copy(rj_v.at[b],
                                          xj_hbm.at[pl.ds(c * CH, CH)],
                                          sem_wb.at[b]))

        def body(g, carry):
            k0 = g * NB
            for b in range(NB):
                c = wid + (k0 + b) * NW

                @pl.when(c < nchunk)
                def _():
                    @pl.when(g > 0)
                    def _():
                        cp, cq = _wb_pair(b, wid + (k0 - NB + b) * NW)
                        cp.wait()
                        cq.wait()
                    pltpu.async_copy(dst2_hbm.at[c], di_v.at[b], sem_ix.at[b])
                    pltpu.async_copy(src2_hbm.at[c], si_v.at[b], sem_ix.at[b])
            for b in range(NB):
                c = wid + (k0 + b) * NW

                @pl.when(c < nchunk)
                def _():
                    pltpu.make_async_copy(dst2_hbm.at[c], di_v.at[b],
                                          sem_ix.at[b]).wait()
                    pltpu.make_async_copy(src2_hbm.at[c], si_v.at[b],
                                          sem_ix.at[b]).wait()
                    pltpu.async_copy(node_hbm.at[di_v.at[b]], ri_v.at[b],
                                     sem_g.at[b])
                    pltpu.async_copy(node_hbm.at[si_v.at[b]], rj_v.at[b],
                                     sem_g.at[b])
            for b in range(NB):
                c = wid + (k0 + b) * NW

                @pl.when(c < nchunk)
                def _():
                    pltpu.make_async_copy(node_hbm.at[di_v.at[b]], ri_v.at[b],
                                          sem_g.at[b]).wait()
                    pltpu.make_async_copy(node_hbm.at[si_v.at[b]], rj_v.at[b],
                                          sem_g.at[b]).wait()
                    cp, cq = _wb_pair(b, c)
                    cp.start()
                    cq.start()
            return carry

        lax.fori_loop(0, ngrp, body, 0)
        for b in range(NB):
            c = wid + ((ngrp - 1) * NB + b) * NW

            @pl.when(c < nchunk)
            def _():
                cp, cq = _wb_pair(b, c)
                cp.wait()
                cq.wait()

    return _sc_gather


@functools.lru_cache(maxsize=None)
def _make_sc_scatter(nchunk):
    ngrp = (nchunk + NW * NB - 1) // (NW * NB)

    @functools.partial(
        pl.kernel,
        out_type=jax.ShapeDtypeStruct((NC, NN, MSGW), jnp.float32),
        mesh=_mesh,
        scratch_types=[
            pltpu.VMEM((NB, CH), jnp.int32),
            pltpu.VMEM((NB, CH, PADF), jnp.float32),
            pltpu.VMEM((NB, CH, MSGW), jnp.float32),
            pltpu.VMEM((ZR, MSGW), jnp.float32),
            pltpu.VMEM_SHARED((NN, MSGW), jnp.float32),
            pltpu.SemaphoreType.DMA((NB,)),
            pltpu.SemaphoreType.DMA((NB,)),
        ],
        compiler_params=_sc_params_nl,
    )
    def _sc_scatter(msg_hbm, dst2_hbm, zrows_hbm, out_hbm, di_v, rm_v, r8_v,
                    zb_v, acc_sh, sem_in, sem_sc):
        cid = lax.axis_index("c")
        sid = lax.axis_index("s")
        wid = sid * NC + cid
        pltpu.sync_copy(zrows_hbm, zb_v)
        for j in range(RPT // ZR):
            pltpu.sync_copy(zb_v, acc_sh.at[pl.ds(sid * RPT + j * ZR, ZR)])
        plsc.subcore_barrier()

        lanes = jnp.arange(16, dtype=jnp.int32)
        rbase = jnp.where(lanes < 8, 0, 1)
        cols = jnp.where(lanes < 8, lanes, lanes - 8)

        def body(g, carry):
            k0 = g * NB
            for b in range(NB):
                c = wid + (k0 + b) * NW

                @pl.when(c < nchunk)
                def _():
                    @pl.when(g > 0)
                    def _():
                        pltpu.make_async_copy(r8_v.at[b],
                                              acc_sh.at[di_v.at[b]],
                                              sem_sc.at[b]).wait()
                    pltpu.async_copy(dst2_hbm.at[c], di_v.at[b], sem_in.at[b])
                    pltpu.async_copy(msg_hbm.at[pl.ds(c * CH, CH)], rm_v.at[b],
                                     sem_in.at[b])
            for b in range(NB):
                c = wid + (k0 + b) * NW

                @pl.when(c < nchunk)
                def _():
                    pltpu.make_async_copy(dst2_hbm.at[c], di_v.at[b],
                                          sem_in.at[b]).wait()
                    pltpu.make_async_copy(msg_hbm.at[pl.ds(c * CH, CH)],
                                          rm_v.at[b], sem_in.at[b]).wait()
                    for j in range(CH // 2):
                        rowv = rbase + (2 * j)
                        v = plsc.load_gather(rm_v.at[b], [rowv, cols])
                        plsc.store_scatter(r8_v.at[b], [rowv, cols], v)
                    pltpu.async_copy(r8_v.at[b], acc_sh.at[di_v.at[b]],
                                     sem_sc.at[b], add=True)
            return carry

        lax.fori_loop(0, ngrp, body, 0)
        for b in range(NB):
            c = wid + ((ngrp - 1) * NB + b) * NW

            @pl.when(c < nchunk)
            def _():
                pltpu.make_async_copy(r8_v.at[b], acc_sh.at[di_v.at[b]],
                                      sem_sc.at[b]).wait()
        plsc.subcore_barrier()
        pltpu.sync_copy(acc_sh.at[pl.ds(sid * RPT, RPT)],
                        out_hbm.at[cid, pl.ds(sid * RPT, RPT)])

    return _sc_scatter


def _full(shape):
    return pl.BlockSpec(shape, lambda i: (0,) * len(shape))


def _mlp2_body(x_ref, w1, b1, w2, b2, o_ref):
    h = jnp.maximum(jnp.dot(x_ref[...], w1[...],
                            preferred_element_type=jnp.float32) + b1[...], 0.0)
    o_ref[...] = jnp.dot(h, w2[...], preferred_element_type=jnp.float32) + b2[...]


def _node_enc_body(x_ref, w1, b1, w2, b2, o_ref):
    h = jnp.maximum(jnp.dot(x_ref[...], w1[...],
                            preferred_element_type=jnp.float32) + b1[...], 0.0)
    v = jnp.dot(h, w2[...], preferred_element_type=jnp.float32) + b2[...]
    o_ref[...] = jnp.concatenate(
        [v, jnp.zeros((v.shape[0], PADF - NFD), jnp.float32)], axis=1)


def _edge_mlp_body(xi_ref, xj_ref, ed_ref, w1i, w1j, w1e, b1, w2, b2,
                   msg_ref, eo_ref):
    ed = ed_ref[...]
    h = (jnp.dot(xi_ref[...], w1i[...], preferred_element_type=jnp.float32)
         + jnp.dot(xj_ref[...], w1j[...], preferred_element_type=jnp.float32)
         + jnp.dot(ed, w1e[...], preferred_element_type=jnp.float32)
         + b1[...])
    h = jnp.maximum(h, 0.0)
    m = jnp.dot(h, w2[...], preferred_element_type=jnp.float32) + b2[...]
    msg_ref[...] = m
    eo_ref[...] = ed + m


def _node_up_body(nd_ref, pp_ref, qq_ref, w1n, w1a, b1, w2, b2, o_ref):
    nd = nd_ref[:, :NFD]
    ag = (pp_ref[0, :, :EFD] + pp_ref[1, :, :EFD]
          + qq_ref[0, :, :EFD] + qq_ref[1, :, :EFD])
    h = jnp.maximum(
        jnp.dot(nd, w1n[...], preferred_element_type=jnp.float32)
        + jnp.dot(ag, w1a[...], preferred_element_type=jnp.float32)
        + b1[...], 0.0)
    v = nd + jnp.dot(h, w2[...], preferred_element_type=jnp.float32) + b2[...]
    o_ref[...] = jnp.concatenate(
        [v, jnp.zeros((v.shape[0], PADF - NFD), jnp.float32)], axis=1)


def _dec_body(nd_ref, w1, b1, w2, b2, o_ref):
    h = jnp.maximum(jnp.dot(nd_ref[:, :NFD], w1[...],
                            preferred_element_type=jnp.float32) + b1[...], 0.0)
    o_ref[...] = jnp.dot(h, w2[...], preferred_element_type=jnp.float32) + b2[...]


def _node_encoder(x, w1, b1, w2, b2):
    return pl.pallas_call(
        _node_enc_body,
        grid=(NN // NBLK,),
        in_specs=[pl.BlockSpec((NBLK, NFD), lambda i: (i, 0)),
                  _full(w1.shape), _full(b1.shape),
                  _full(w2.shape), _full(b2.shape)],
        out_specs=pl.BlockSpec((NBLK, PADF), lambda i: (i, 0)),
        out_shape=jax.ShapeDtypeStruct((NN, PADF), jnp.float32),
    )(x, w1, b1, w2, b2)


def _edge_encoder(ea4, w1, b1, w2, b2):
    rows = ea4.shape[0]
    return pl.pallas_call(
        _mlp2_body,
        grid=(rows // EB4,),
        in_specs=[pl.BlockSpec((EB4, 128), lambda i: (i, 0)),
                  _full(w1.shape), _full(b1.shape),
                  _full(w2.shape), _full(b2.shape)],
        out_specs=pl.BlockSpec((EB4, 128), lambda i: (i, 0)),
        out_shape=jax.ShapeDtypeStruct((rows, 128), jnp.float32),
    )(ea4, w1, b1, w2, b2)


def _edge_mlp(xi4, xj4, ed4, w1i, w1j, w1e, b1, w2, b2):
    rows = ed4.shape[0]
    eb = pl.BlockSpec((EB4, 128), lambda i: (i, 0))
    return pl.pallas_call(
        _edge_mlp_body,
        grid=(rows // EB4,),
        in_specs=[eb, eb, eb,
                  _full(w1i.shape), _full(w1j.shape), _full(w1e.shape),
                  _full(b1.shape), _full(w2.shape), _full(b2.shape)],
        out_specs=[eb, eb],
        out_shape=[jax.ShapeDtypeStruct((rows, 128), jnp.float32),
                   jax.ShapeDtypeStruct((rows, 128), jnp.float32)],
    )(xi4, xj4, ed4, w1i, w1j, w1e, b1, w2, b2)


def _node_update(node, parts0, parts1, w1n, w1a, b1, w2, b2):
    pspec = pl.BlockSpec((NC, NBLK, MSGW), lambda i: (0, i, 0))
    return pl.pallas_call(
        _node_up_body,
        grid=(NN // NBLK,),
        in_specs=[pl.BlockSpec((NBLK, PADF), lambda i: (i, 0)),
                  pspec, pspec,
                  _full(w1n.shape), _full(w1a.shape), _full(b1.shape),
                  _full(w2.shape), _full(b2.shape)],
        out_specs=pl.BlockSpec((NBLK, PADF), lambda i: (i, 0)),
        out_shape=jax.ShapeDtypeStruct((NN, PADF), jnp.float32),
    )(node, parts0, parts1, w1n, w1a, b1, w2, b2)


def _decoder(node, w1, b1, w2, b2):
    return pl.pallas_call(
        _dec_body,
        grid=(NN // NBLK,),
        in_specs=[pl.BlockSpec((NBLK, PADF), lambda i: (i, 0)),
                  _full(w1.shape), _full(b1.shape),
                  _full(w2.shape), _full(b2.shape)],
        out_specs=pl.BlockSpec((NBLK, EFD), lambda i: (i, 0)),
        out_shape=jax.ShapeDtypeStruct((NN, EFD), jnp.float32),
    )(node, w1, b1, w2, b2)


def _bd(w, rows=PADF, cols=PADF):
    wp = jnp.zeros((rows, cols), jnp.float32).at[:w.shape[0], :w.shape[1]].set(w)
    return jnp.kron(jnp.eye(G, dtype=jnp.float32), wp)


def _tile_bias(b, width=PADF):
    bp = jnp.zeros((width,), jnp.float32).at[:b.shape[0]].set(b)
    return jnp.tile(bp, G).reshape(1, G * width)


NSPLIT = 2
NCH_H = NCHUNK // NSPLIT
EE_H = EE // NSPLIT
E4_H = E4 // NSPLIT


def kernel(x, edge_index, edge_attr, enW1, enb1, enW2, enb2,
           eeW1, eeb1, eeW2, eeb2, peW1, peb1, peW2, peb2,
           pnW1, pnb1, pnW2, pnb2, dW1, db1, dW2, db2):
    src = edge_index[0]
    dst = edge_index[1]
    dst2 = dst.reshape(NCHUNK, CH)
    src2 = src.reshape(NCHUNK, CH)
    dsth = [dst2[k * NCH_H:(k + 1) * NCH_H] for k in range(NSPLIT)]
    srch = [src2[k * NCH_H:(k + 1) * NCH_H] for k in range(NSPLIT)]
    zrows = jnp.zeros((ZR, MSGW), jnp.float32)
    gath = _make_sc_gather(NCH_H)
    scat = _make_sc_scatter(NCH_H)

    node = _node_encoder(x, enW1, enb1.reshape(1, -1), enW2, enb2.reshape(1, -1))

    ea4 = jnp.pad(edge_attr.reshape(E4, G, EFD),
                  ((0, 0), (0, 0), (0, PADF - EFD))).reshape(E4, G * PADF)
    edgeh = [_edge_encoder(ea4[k * E4_H:(k + 1) * E4_H],
                           _bd(eeW1, PADF, 64), _tile_bias(eeb1, 64),
                           _bd(eeW2, 64, PADF), _tile_bias(eeb2))
             for k in range(NSPLIT)]

    for i in range(NLAYER):
        gh = [gath(node, dsth[k], srch[k]) for k in range(NSPLIT)]
        parth = []
        for k in range(NSPLIT):
            xi, xj = gh[k]
            msg4, edgeh[k] = _edge_mlp(
                xi.reshape(E4_H, 128), xj.reshape(E4_H, 128), edgeh[k],
                _bd(peW1[i][:NFD]), _bd(peW1[i][NFD:2 * NFD]),
                _bd(peW1[i][2 * NFD:]), _tile_bias(peb1[i]),
                _bd(peW2[i]), _tile_bias(peb2[i]))
            parth.append(scat(msg4.reshape(EE_H, PADF), dsth[k], zrows))
        node = _node_update(node, parth[0], parth[1],
                            pnW1[i][:NFD], pnW1[i][NFD:],
                            pnb1[i].reshape(1, -1), pnW2[i],
                            pnb2[i].reshape(1, -1))

    return _decoder(node, dW1, db1.reshape(1, -1), dW2, db2.reshape(1, -1))


# speedup vs baseline: 1.0716x; 1.0716x over previous
import functools

import jax
import jax.numpy as jnp
from jax import lax
from jax.experimental import pallas as pl
from jax.experimental.pallas import tpu as pltpu
from jax.experimental.pallas import tpu_sc as plsc

NN = 100000
EE = 3200000
NFD = 18
EFD = 3
NLAYER = 10

PADF = 32
MSGW = 8
G = 4
E4 = EE // G
G8 = 16
E16 = EE // G8

NC, NS = 2, 16
NW = NC * NS
CH = 128
NCHUNK = EE // CH
RPT = NN // NS
ZR = 1250

NBLK = 1000
EB4 = 800
EB16 = EB4 // 4

_mesh = plsc.VectorSubcoreMesh(core_axis_name="c", subcore_axis_name="s",
                               num_cores=NC, num_subcores=NS)
_sc_params = pltpu.CompilerParams(use_tc_tiling_on_sc=False)

NB = 8


@functools.lru_cache(maxsize=None)
def _make_sc_gather(nchunk):
    ngrp = (nchunk + NW * NB - 1) // (NW * NB)

    @functools.partial(
        pl.kernel,
        out_type=(jax.ShapeDtypeStruct((nchunk * CH, PADF), jnp.float32),
                  jax.ShapeDtypeStruct((nchunk * CH, PADF), jnp.float32)),
        mesh=_mesh,
        scratch_types=[
            pltpu.VMEM((NB, CH), jnp.int32),
            pltpu.VMEM((NB, CH), jnp.int32),
            pltpu.VMEM((NB, CH, PADF), jnp.float32),
            pltpu.VMEM((NB, CH, PADF), jnp.float32),
            pltpu.SemaphoreType.DMA((NB,)),
            pltpu.SemaphoreType.DMA((NB,)),
            pltpu.SemaphoreType.DMA((NB,)),
        ],
        compiler_params=_sc_params,
    )
    def _sc_gather(node_hbm, dst2_hbm, src2_hbm, xi_hbm, xj_hbm,
                   di_v, si_v, ri_v, rj_v, sem_ix, sem_g, sem_wb):
        wid = lax.axis_index("s") * NC + lax.axis_index("c")

        def _wb_pair(b, c):
            return (pltpu.make_async_copy(ri_v.at[b],
                                          xi_hbm.at[pl.ds(c * CH, CH)],
                                          sem_wb.at[b]),
                    pltpu.make_async_copy(rj_v.at[b],
                                          xj_hbm.at[pl.ds(c * CH, CH)],
                                          sem_wb.at[b]))

        def body(g, carry):
            k0 = g * NB
            for b in range(NB):
                c = wid + (k0 + b) * NW

                @pl.when(c < nchunk)
                def _():
                    @pl.when(g > 0)
                    def _():
                        cp, cq = _wb_pair(b, wid + (k0 - NB + b) * NW)
                        cp.wait()
                        cq.wait()
                    pltpu.async_copy(dst2_hbm.at[c], di_v.at[b], sem_ix.at[b])
                    pltpu.async_copy(src2_hbm.at[c], si_v.at[b], sem_ix.at[b])
            for b in range(NB):
                c = wid + (k0 + b) * NW

                @pl.when(c < nchunk)
                def _():
                    pltpu.make_async_copy(dst2_hbm.at[c], di_v.at[b],
                                          sem_ix.at[b]).wait()
                    pltpu.make_async_copy(src2_hbm.at[c], si_v.at[b],
                                          sem_ix.at[b]).wait()
                    pltpu.async_copy(node_hbm.at[di_v.at[b]], ri_v.at[b],
                                     sem_g.at[b])
                    pltpu.async_copy(node_hbm.at[si_v.at[b]], rj_v.at[b],
                                     sem_g.at[b])
            for b in range(NB):
                c = wid + (k0 + b) * NW

                @pl.when(c < nchunk)
                def _():
                    pltpu.make_async_copy(node_hbm.at[di_v.at[b]], ri_v.at[b],
                                          sem_g.at[b]).wait()
                    pltpu.make_async_copy(node_hbm.at[si_v.at[b]], rj_v.at[b],
                                          sem_g.at[b]).wait()
                    cp, cq = _wb_pair(b, c)
                    cp.start()
                    cq.start()
            return carry

        lax.fori_loop(0, ngrp, body, 0)
        for b in range(NB):
            c = wid + ((ngrp - 1) * NB + b) * NW

            @pl.when(c < nchunk)
            def _():
                cp, cq = _wb_pair(b, c)
                cp.wait()
                cq.wait()

    return _sc_gather


@functools.lru_cache(maxsize=None)
def _make_sc_scatter(nchunk):
    ngrp = (nchunk + NW * NB - 1) // (NW * NB)

    @functools.partial(
        pl.kernel,
        out_type=jax.ShapeDtypeStruct((NC, NN, MSGW), jnp.float32),
        mesh=_mesh,
        scratch_types=[
            pltpu.VMEM((NB, CH), jnp.int32),
            pltpu.VMEM((NB, CH, MSGW), jnp.float32),
            pltpu.VMEM((ZR, MSGW), jnp.float32),
            pltpu.VMEM_SHARED((NN, MSGW), jnp.float32),
            pltpu.SemaphoreType.DMA((NB,)),
            pltpu.SemaphoreType.DMA((NB,)),
        ],
        compiler_params=_sc_params,
    )
    def _sc_scatter(msg_hbm, dst2_hbm, zrows_hbm, out_hbm, di_v, r8_v,
                    zb_v, acc_sh, sem_in, sem_sc):
        cid = lax.axis_index("c")
        sid = lax.axis_index("s")
        wid = sid * NC + cid
        pltpu.sync_copy(zrows_hbm, zb_v)
        for j in range(RPT // ZR):
            pltpu.sync_copy(zb_v, acc_sh.at[pl.ds(sid * RPT + j * ZR, ZR)])
        plsc.subcore_barrier()

        def body(g, carry):
            k0 = g * NB
            for b in range(NB):
                c = wid + (k0 + b) * NW

                @pl.when(c < nchunk)
                def _():
                    @pl.when(g > 0)
                    def _():
                        pltpu.make_async_copy(r8_v.at[b],
                                              acc_sh.at[di_v.at[b]],
                                              sem_sc.at[b]).wait()
                    pltpu.async_copy(dst2_hbm.at[c], di_v.at[b], sem_in.at[b])
                    pltpu.async_copy(msg_hbm.at[pl.ds(c * CH, CH)], r8_v.at[b],
                                     sem_in.at[b])
            for b in range(NB):
                c = wid + (k0 + b) * NW

                @pl.when(c < nchunk)
                def _():
                    pltpu.make_async_copy(dst2_hbm.at[c], di_v.at[b],
                                          sem_in.at[b]).wait()
                    pltpu.make_async_copy(msg_hbm.at[pl.ds(c * CH, CH)],
                                          r8_v.at[b], sem_in.at[b]).wait()
                    pltpu.async_copy(r8_v.at[b], acc_sh.at[di_v.at[b]],
                                     sem_sc.at[b], add=True)
            return carry

        lax.fori_loop(0, ngrp, body, 0)
        for b in range(NB):
            c = wid + ((ngrp - 1) * NB + b) * NW

            @pl.when(c < nchunk)
            def _():
                pltpu.make_async_copy(r8_v.at[b], acc_sh.at[di_v.at[b]],
                                      sem_sc.at[b]).wait()
        plsc.subcore_barrier()
        pltpu.sync_copy(acc_sh.at[pl.ds(sid * RPT, RPT)],
                        out_hbm.at[cid, pl.ds(sid * RPT, RPT)])

    return _sc_scatter


def _full(shape):
    return pl.BlockSpec(shape, lambda i: (0,) * len(shape))


def _edge_enc_body(x_ref, w1, b1, w2, b2, p32, o_ref):
    h = jnp.maximum(jnp.dot(x_ref[...], w1[...],
                            preferred_element_type=jnp.float32) + b1[...], 0.0)
    v = jnp.dot(h, w2[...], preferred_element_type=jnp.float32) + b2[...]
    v4 = v.reshape(EB16, 4, 128)
    o_ref[...] = jnp.concatenate(
        [jnp.dot(v4[:, j], p32[...], preferred_element_type=jnp.float32)
         for j in range(4)], axis=1)


def _node_enc_body(x_ref, w1, b1, w2, b2, o_ref):
    h = jnp.maximum(jnp.dot(x_ref[...], w1[...],
                            preferred_element_type=jnp.float32) + b1[...], 0.0)
    v = jnp.dot(h, w2[...], preferred_element_type=jnp.float32) + b2[...]
    o_ref[...] = jnp.concatenate(
        [v, jnp.zeros((v.shape[0], PADF - NFD), jnp.float32)], axis=1)


def _edge_mlp_body(xi_ref, xj_ref, ed8_ref, w1i, w1j, w1e8, b1, w2c, b2t,
                   msg_ref, eo_ref):
    ed8 = ed8_ref[...]
    edc = jnp.stack(
        [jnp.dot(ed8[:, 32 * j:32 * (j + 1)], w1e8[...],
                 preferred_element_type=jnp.float32) for j in range(4)],
        axis=1).reshape(EB4, 128)
    h = (jnp.dot(xi_ref[...], w1i[...], preferred_element_type=jnp.float32)
         + jnp.dot(xj_ref[...], w1j[...], preferred_element_type=jnp.float32)
         + edc + b1[...])
    h = jnp.maximum(h, 0.0)
    h4 = h.reshape(EB16, 4, 128)
    m = jnp.concatenate(
        [jnp.dot(h4[:, j], w2c[...], preferred_element_type=jnp.float32)
         for j in range(4)], axis=1) + b2t[...]
    msg_ref[...] = m
    eo_ref[...] = ed8 + m


def _node_up_body(nd_ref, pp_ref, qq_ref, w1n, w1a, b1, w2, b2, o_ref):
    nd = nd_ref[:, :NFD]
    ag = (pp_ref[0, :, :EFD] + pp_ref[1, :, :EFD]
          + qq_ref[0, :, :EFD] + qq_ref[1, :, :EFD])
    h = jnp.maximum(
        jnp.dot(nd, w1n[...], preferred_element_type=jnp.float32)
        + jnp.dot(ag, w1a[...], preferred_element_type=jnp.float32)
        + b1[...], 0.0)
    v = nd + jnp.dot(h, w2[...], preferred_element_type=jnp.float32) + b2[...]
    o_ref[...] = jnp.concatenate(
        [v, jnp.zeros((v.shape[0], PADF - NFD), jnp.float32)], axis=1)


def _dec_body(nd_ref, w1, b1, w2, b2, o_ref):
    h = jnp.maximum(jnp.dot(nd_ref[:, :NFD], w1[...],
                            preferred_element_type=jnp.float32) + b1[...], 0.0)
    o_ref[...] = jnp.dot(h, w2[...], preferred_element_type=jnp.float32) + b2[...]


def _node_encoder(x, w1, b1, w2, b2):
    return pl.pallas_call(
        _node_enc_body,
        grid=(NN // NBLK,),
        in_specs=[pl.BlockSpec((NBLK, NFD), lambda i: (i, 0)),
                  _full(w1.shape), _full(b1.shape),
                  _full(w2.shape), _full(b2.shape)],
        out_specs=pl.BlockSpec((NBLK, PADF), lambda i: (i, 0)),
        out_shape=jax.ShapeDtypeStruct((NN, PADF), jnp.float32),
    )(x, w1, b1, w2, b2)


def _edge_encoder(ea4, w1, b1, w2, b2, p32):
    rows = ea4.shape[0]
    return pl.pallas_call(
        _edge_enc_body,
        grid=(rows // EB4,),
        in_specs=[pl.BlockSpec((EB4, 128), lambda i: (i, 0)),
                  _full(w1.shape), _full(b1.shape),
                  _full(w2.shape), _full(b2.shape), _full(p32.shape)],
        out_specs=pl.BlockSpec((EB16, 128), lambda i: (i, 0)),
        out_shape=jax.ShapeDtypeStruct((rows // 4, 128), jnp.float32),
    )(ea4, w1, b1, w2, b2, p32)


def _edge_mlp(xi4, xj4, ed8, w1i, w1j, w1e8, b1, w2c, b2t):
    rows = xi4.shape[0]
    eb = pl.BlockSpec((EB4, 128), lambda i: (i, 0))
    e8 = pl.BlockSpec((EB16, 128), lambda i: (i, 0))
    return pl.pallas_call(
        _edge_mlp_body,
        grid=(rows // EB4,),
        in_specs=[eb, eb, e8,
                  _full(w1i.shape), _full(w1j.shape), _full(w1e8.shape),
                  _full(b1.shape), _full(w2c.shape), _full(b2t.shape)],
        out_specs=[e8, e8],
        out_shape=[jax.ShapeDtypeStruct((rows // 4, 128), jnp.float32),
                   jax.ShapeDtypeStruct((rows // 4, 128), jnp.float32)],
    )(xi4, xj4, ed8, w1i, w1j, w1e8, b1, w2c, b2t)


def _node_update(node, parts0, parts1, w1n, w1a, b1, w2, b2):
    pspec = pl.BlockSpec((NC, NBLK, MSGW), lambda i: (0, i, 0))
    return pl.pallas_call(
        _node_up_body,
        grid=(NN // NBLK,),
        in_specs=[pl.BlockSpec((NBLK, PADF), lambda i: (i, 0)),
                  pspec, pspec,
                  _full(w1n.shape), _full(w1a.shape), _full(b1.shape),
                  _full(w2.shape), _full(b2.shape)],
        out_specs=pl.BlockSpec((NBLK, PADF), lambda i: (i, 0)),
        out_shape=jax.ShapeDtypeStruct((NN, PADF), jnp.float32),
    )(node, parts0, parts1, w1n, w1a, b1, w2, b2)


def _decoder(node, w1, b1, w2, b2):
    return pl.pallas_call(
        _dec_body,
        grid=(NN // NBLK,),
        in_specs=[pl.BlockSpec((NBLK, PADF), lambda i: (i, 0)),
                  _full(w1.shape), _full(b1.shape),
                  _full(w2.shape), _full(b2.shape)],
        out_specs=pl.BlockSpec((NBLK, EFD), lambda i: (i, 0)),
        out_shape=jax.ShapeDtypeStruct((NN, EFD), jnp.float32),
    )(node, w1, b1, w2, b2)


def _bd(w, rows=PADF, cols=PADF):
    wp = jnp.zeros((rows, cols), jnp.float32).at[:w.shape[0], :w.shape[1]].set(w)
    return jnp.kron(jnp.eye(G, dtype=jnp.float32), wp)


def _tile_bias(b, width=PADF, reps=G):
    bp = jnp.zeros((width,), jnp.float32).at[:b.shape[0]].set(b)
    return jnp.tile(bp, reps).reshape(1, reps * width)


NSPLIT = 2
NCH_H = NCHUNK // NSPLIT
EE_H = EE // NSPLIT
E4_H = E4 // NSPLIT


def kernel(x, edge_index, edge_attr, enW1, enb1, enW2, enb2,
           eeW1, eeb1, eeW2, eeb2, peW1, peb1, peW2, peb2,
           pnW1, pnb1, pnW2, pnb2, dW1, db1, dW2, db2):
    src = edge_index[0]
    dst = edge_index[1]
    dst2 = dst.reshape(NCHUNK, CH)
    src2 = src.reshape(NCHUNK, CH)
    dsth = [dst2[k * NCH_H:(k + 1) * NCH_H] for k in range(NSPLIT)]
    srch = [src2[k * NCH_H:(k + 1) * NCH_H] for k in range(NSPLIT)]
    zrows = jnp.zeros((ZR, MSGW), jnp.float32)
    gath = _make_sc_gather(NCH_H)
    scat = _make_sc_scatter(NCH_H)

    node = _node_encoder(x, enW1, enb1.reshape(1, -1), enW2, enb2.reshape(1, -1))

    ea4 = jnp.pad(edge_attr.reshape(E4, G, EFD),
                  ((0, 0), (0, 0), (0, PADF - EFD))).reshape(E4, G * PADF)
    p32 = _bd(jnp.eye(PADF, MSGW, dtype=jnp.float32), PADF, MSGW)
    edgeh = [_edge_encoder(ea4[k * E4_H:(k + 1) * E4_H],
                           _bd(eeW1, PADF, 64), _tile_bias(eeb1, 64),
                           _bd(eeW2, 64, PADF), _tile_bias(eeb2), p32)
             for k in range(NSPLIT)]

    for i in range(NLAYER):
        gh = [gath(node, dsth[k], srch[k]) for k in range(NSPLIT)]
        parth = []
        for k in range(NSPLIT):
            xi, xj = gh[k]
            msg8, edgeh[k] = _edge_mlp(
                xi.reshape(E4_H, 128), xj.reshape(E4_H, 128), edgeh[k],
                _bd(peW1[i][:NFD]), _bd(peW1[i][NFD:2 * NFD]),
                _bd(peW1[i][2 * NFD:], MSGW, PADF), _tile_bias(peb1[i]),
                _bd(peW2[i], PADF, MSGW), _tile_bias(peb2[i], MSGW, G8))
            parth.append(scat(msg8.reshape(EE_H, MSGW), dsth[k], zrows))
        node = _node_update(node, parth[0], parth[1],
                            pnW1[i][:NFD], pnW1[i][NFD:],
                            pnb1[i].reshape(1, -1), pnW2[i],
                            pnb2[i].reshape(1, -1))

    return _decoder(node, dW1, db1.reshape(1, -1), dW2, db2.reshape(1, -1))
